# baseline (speedup 1.0000x reference)
import functools

import jax
import jax.numpy as jnp
from jax import lax
from jax.experimental import pallas as pl
from jax.experimental.pallas import tpu as pltpu
from jax.experimental.pallas import tpu_sc as plsc

VOCAB = 1000000
N_EMBD = 64
B, L = 4096, 200

G = 128
N = B * L
NROWS = N // G
NW = 32
ROWS_PER_W = NROWS // NW
K = 4
STAGES = ROWS_PER_W // K


def _gather_body(idx_hbm, table_hbm, out_hbm, idx_v, rows_v, sem):
    c = lax.axis_index("c")
    s = lax.axis_index("s")
    wid = s * 2 + c
    base = wid * ROWS_PER_W

    @pl.loop(0, STAGES)
    def _stage(g):
        row = base + g * K
        pltpu.sync_copy(idx_hbm.at[pl.ds(row, K)], idx_v)
        copies = [
            pltpu.async_copy(table_hbm.at[idx_v.at[j]], rows_v.at[j], sem)
            for j in range(K)
        ]
        for cp in copies:
            cp.wait()
        pltpu.sync_copy(rows_v, out_hbm.at[pl.ds(row, K)])


@jax.jit
def _embed_lookup(xf, token_embed):
    mesh = plsc.VectorSubcoreMesh(core_axis_name="c", subcore_axis_name="s")
    return pl.kernel(
        _gather_body,
        out_type=jax.ShapeDtypeStruct((NROWS, G, N_EMBD), jnp.float32),
        mesh=mesh,
        scratch_types=[
            pltpu.VMEM((K, G), jnp.int32),
            pltpu.VMEM((K, G, N_EMBD), jnp.float32),
            pltpu.SemaphoreType.DMA,
        ],
        compiler_params=pltpu.CompilerParams(use_tc_tiling_on_sc=False),
    )(xf, token_embed)


def kernel(x, token_embed):
    xf = x.reshape(NROWS, G).astype(jnp.int32)
    out = _embed_lookup(xf, token_embed)
    return out.reshape(B, L, N_EMBD)


# speedup vs baseline: 1.0429x; 1.0429x over previous
import jax
import jax.numpy as jnp
from jax import lax
from jax.experimental import pallas as pl
from jax.experimental.pallas import tpu as pltpu
from jax.experimental.pallas import tpu_sc as plsc

VOCAB = 1000000
N_EMBD = 64
B, L = 4096, 200

G = 128
N = B * L
NROWS = N // G
NW = 32
ROWS_PER_W = NROWS // NW
K = 4
STAGES = ROWS_PER_W // K
NBUF = 2


def _gather_body(idx_hbm, table_hbm, out_hbm, idx_v, rows_v, gsems, osems):
    c = lax.axis_index("c")
    s = lax.axis_index("s")
    wid = s * 2 + c
    base = wid * ROWS_PER_W

    pltpu.sync_copy(idx_hbm.at[pl.ds(base, ROWS_PER_W)], idx_v)

    def fire_gathers(b, stage):
        for j in range(K):
            pltpu.async_copy(
                table_hbm.at[idx_v.at[stage * K + j]],
                rows_v.at[b].at[j],
                gsems[b],
            )

    def drain(sem, row, b):
        pltpu.make_async_copy(
            out_hbm.at[pl.ds(row, K)], rows_v.at[b], sem
        ).wait()

    for b in range(NBUF):
        fire_gathers(b, b)

    @pl.loop(0, (STAGES - NBUF) // NBUF)
    def _t(t):
        for b in range(NBUF):
            stage = t * NBUF + b
            row = base + stage * K
            drain(gsems[b], row, b)
            cp = pltpu.async_copy(rows_v.at[b], out_hbm.at[pl.ds(row, K)], osems[b])
            cp.wait()
            fire_gathers(b, stage + NBUF)

    for b in range(NBUF):
        stage_e = STAGES - NBUF + b
        row_e = base + stage_e * K
        drain(gsems[b], row_e, b)
        pltpu.async_copy(rows_v.at[b], out_hbm.at[pl.ds(row_e, K)], osems[b]).wait()


@jax.jit
def _embed_lookup(xf, token_embed):
    mesh = plsc.VectorSubcoreMesh(core_axis_name="c", subcore_axis_name="s")
    return pl.kernel(
        _gather_body,
        out_type=jax.ShapeDtypeStruct((NROWS, G, N_EMBD), jnp.float32),
        mesh=mesh,
        scratch_types=[
            pltpu.VMEM((ROWS_PER_W, G), jnp.int32),
            pltpu.VMEM((NBUF, K, G, N_EMBD), jnp.float32),
            [pltpu.SemaphoreType.DMA] * NBUF,
            [pltpu.SemaphoreType.DMA] * NBUF,
        ],
        compiler_params=pltpu.CompilerParams(use_tc_tiling_on_sc=False),
    )(xf, token_embed)


def kernel(x, token_embed):
    xf = x.reshape(NROWS, G).astype(jnp.int32)
    out = _embed_lookup(xf, token_embed)
    return out.reshape(B, L, N_EMBD)
